# baseline (speedup 1.0000x reference)
import functools

import jax
import jax.numpy as jnp
from jax import lax
from jax.experimental import pallas as pl
from jax.experimental.pallas import tpu as pltpu
from jax.experimental.pallas import tpu_sc as plsc

_B, _W = 16384, 100
_NC, _NS, _L = 2, 16, 16
_NW = _NC * _NS
_RPW = _B // _NW
_GROUPS = _RPW // _L

_TC_GRID = 8
_TC_ROWS = _B // _TC_GRID
_OUT_R = 128
_SLAB_R = _RPW // _OUT_R


def _tc_stats_body(pred_ref, a_ref, val_ref):
    x = pred_ref[...]
    rev_iota = jnp.int32(127) - lax.broadcasted_iota(
        jnp.int32, (_TC_ROWS, _W), 1)

    b = lax.bitcast_convert_type(x, jnp.int32)
    mask = (b >> 31) & jnp.int32(0x7FFFFFFF)
    keys = ((b ^ mask) & jnp.int32(~127)) | rev_iota

    kp = jnp.max(keys, axis=1)
    a = jnp.int32(127) - (kp & jnp.int32(127))
    minv = kp ^ ((kp >> 31) & jnp.int32(0x7FFFFFFF))
    m = lax.bitcast_convert_type(minv, jnp.float32)

    s = jnp.sum(jnp.exp(x), axis=1)
    val = jnp.exp(m) / s

    rows = _TC_ROWS // _OUT_R
    a_ref[...] = a.reshape(rows, _OUT_R)
    val_ref[...] = val.reshape(rows, _OUT_R)


def _make_tc_stats():
    rows = _TC_ROWS // _OUT_R
    return pl.pallas_call(
        _tc_stats_body,
        grid=(_TC_GRID,),
        in_specs=[pl.BlockSpec((_TC_ROWS, _W), lambda g: (g, 0))],
        out_specs=[
            pl.BlockSpec((rows, _OUT_R), lambda g: (g, 0)),
            pl.BlockSpec((rows, _OUT_R), lambda g: (g, 0)),
        ],
        out_shape=[
            jax.ShapeDtypeStruct((_OUT_R, _OUT_R), jnp.int32),
            jax.ShapeDtypeStruct((_OUT_R, _OUT_R), jnp.float32),
        ],
    )


def _make_sc_stage():
    mesh = plsc.VectorSubcoreMesh(
        core_axis_name="c", subcore_axis_name="s",
        num_cores=_NC, num_subcores=_NS)

    @functools.partial(
        pl.kernel,
        mesh=mesh,
        compiler_params=pltpu.CompilerParams(needs_layout_passes=False),
        out_type=jax.ShapeDtypeStruct((_NW, _L), jnp.float32),
        scratch_types=[
            pltpu.VMEM((_RPW * _W,), jnp.float32),
            pltpu.VMEM((_SLAB_R, _OUT_R), jnp.int32),
            pltpu.VMEM((_SLAB_R, _OUT_R), jnp.float32),
            pltpu.VMEM((_W, _W), jnp.float32),
            pltpu.VMEM((_L,), jnp.float32),
            pltpu.SemaphoreType.DMA,
            pltpu.SemaphoreType.DMA,
            pltpu.SemaphoreType.DMA,
            pltpu.SemaphoreType.DMA,
        ],
    )
    def sc_stage(targ_hbm, a_hbm, val_hbm, pm_hbm, out_hbm,
                 targ_v, a_v, val_v, pm_v, acc_v, sem0, sem1, sem2, sem3):
        wid = lax.axis_index("s") * _NC + lax.axis_index("c")
        h0 = pltpu.async_copy(
            targ_hbm.at[pl.ds(wid * (_RPW * _W), _RPW * _W)], targ_v, sem0)
        r0 = wid * _SLAB_R
        h1 = pltpu.async_copy(a_hbm.at[pl.ds(r0, _SLAB_R)], a_v, sem1)
        h2 = pltpu.async_copy(val_hbm.at[pl.ds(r0, _SLAB_R)], val_v, sem2)
        h3 = pltpu.async_copy(pm_hbm, pm_v, sem3)
        h0.wait()
        h1.wait()
        h2.wait()
        h3.wait()

        lanes = lax.iota(jnp.int32, _L)
        zero_f = jnp.zeros((_L,), jnp.float32)
        zero_i = jnp.zeros((_L,), jnp.int32)
        neg_inf = jnp.full((_L,), -jnp.inf, jnp.float32)

        def group_body(g, acc):
            rowoff = (g * _L + lanes) * _W

            tm, t = neg_inf, zero_i
            for j in range(_W):
                q = plsc.load_gather(targ_v, [rowoff + j])
                upd = q > tm
                tm = jnp.where(upd, q, tm)
                t = jnp.where(upd, jnp.full((_L,), j, jnp.int32), t)

            row = g // (_OUT_R // _L)
            col = (g % (_OUT_R // _L)) * _L
            av = a_v[row, pl.ds(col, _L)]
            vv = val_v[row, pl.ds(col, _L)]
            pmv = plsc.load_gather(pm_v, [t, av])
            return acc + jnp.where(av != t, pmv * vv, zero_f)

        acc = lax.fori_loop(0, _GROUPS, group_body, zero_f)
        acc_v[...] = acc
        pltpu.sync_copy(acc_v, out_hbm.at[wid])

    return sc_stage


_CALLS = {}


def kernel(predict, target, penalty_matrix):
    if not _CALLS:
        _CALLS["tc"] = _make_tc_stats()
        _CALLS["sc"] = jax.jit(_make_sc_stage())
    a, val = _CALLS["tc"](predict)
    partials = _CALLS["sc"](target.reshape(-1), a, val, penalty_matrix)
    return jnp.sum(partials) / jnp.float32(predict.shape[0])


# speedup vs baseline: 1.2499x; 1.2499x over previous
import functools

import jax
import jax.numpy as jnp
from jax import lax
from jax.experimental import pallas as pl
from jax.experimental.pallas import tpu as pltpu
from jax.experimental.pallas import tpu_sc as plsc

_B, _W = 16384, 100
_NC, _NS, _L = 2, 16, 16
_NW = _NC * _NS
_RPW = _B // _NW

_TC_GRID = 8
_TC_ROWS = _B // _TC_GRID
_OUT_R = 128


def _pack_keys(v, rev_iota):
    b = lax.bitcast_convert_type(v, jnp.int32)
    mask = (b >> 31) & jnp.int32(0x7FFFFFFF)
    return ((b ^ mask) & jnp.int32(~127)) | rev_iota


def _tc_stats_body(pred_ref, targ_ref, a_ref, t_ref, val_ref):
    x = pred_ref[...]
    y = targ_ref[...]
    rev_iota = jnp.int32(127) - lax.broadcasted_iota(
        jnp.int32, (_TC_ROWS, _W), 1)

    kp = jnp.max(_pack_keys(x, rev_iota), axis=1)
    a = jnp.int32(127) - (kp & jnp.int32(127))
    minv = kp ^ ((kp >> 31) & jnp.int32(0x7FFFFFFF))
    m = lax.bitcast_convert_type(minv, jnp.float32)

    s = jnp.sum(jnp.exp(x), axis=1)

    kt = jnp.max(_pack_keys(y, rev_iota), axis=1)
    t = jnp.int32(127) - (kt & jnp.int32(127))

    val = jnp.where(a != t, jnp.exp(m) / s, jnp.float32(0.0))

    rows = _TC_ROWS // _OUT_R
    a_ref[...] = a.reshape(rows, _OUT_R)
    t_ref[...] = t.reshape(rows, _OUT_R)
    val_ref[...] = val.reshape(rows, _OUT_R)


def _make_tc_stats():
    rows = _TC_ROWS // _OUT_R
    return pl.pallas_call(
        _tc_stats_body,
        grid=(_TC_GRID,),
        in_specs=[
            pl.BlockSpec((_TC_ROWS, _W), lambda g: (g, 0)),
            pl.BlockSpec((_TC_ROWS, _W), lambda g: (g, 0)),
        ],
        out_specs=[
            pl.BlockSpec((rows, _OUT_R), lambda g: (g, 0)),
            pl.BlockSpec((rows, _OUT_R), lambda g: (g, 0)),
            pl.BlockSpec((rows, _OUT_R), lambda g: (g, 0)),
        ],
        out_shape=[
            jax.ShapeDtypeStruct((_OUT_R, _OUT_R), jnp.int32),
            jax.ShapeDtypeStruct((_OUT_R, _OUT_R), jnp.int32),
            jax.ShapeDtypeStruct((_OUT_R, _OUT_R), jnp.float32),
        ],
    )


_SLAB_R = _RPW // _OUT_R


def _make_sc_gather():
    mesh = plsc.VectorSubcoreMesh(
        core_axis_name="c", subcore_axis_name="s",
        num_cores=_NC, num_subcores=_NS)

    @functools.partial(
        pl.kernel,
        mesh=mesh,
        compiler_params=pltpu.CompilerParams(needs_layout_passes=False),
        out_type=jax.ShapeDtypeStruct((_NW, _L), jnp.float32),
        scratch_types=[
            pltpu.VMEM((_SLAB_R, _OUT_R), jnp.int32),
            pltpu.VMEM((_SLAB_R, _OUT_R), jnp.int32),
            pltpu.VMEM((_SLAB_R, _OUT_R), jnp.float32),
            pltpu.VMEM((_W * _W,), jnp.float32),
            pltpu.VMEM((_L,), jnp.float32),
            pltpu.SemaphoreType.DMA,
            pltpu.SemaphoreType.DMA,
            pltpu.SemaphoreType.DMA,
            pltpu.SemaphoreType.DMA,
        ],
    )
    def sc_gather(a_hbm, t_hbm, val_hbm, pm_hbm, out_hbm,
                  a_v, t_v, val_v, pm_v, acc_v, sem0, sem1, sem2, sem3):
        wid = lax.axis_index("s") * _NC + lax.axis_index("c")
        r0 = wid * _SLAB_R
        h0 = pltpu.async_copy(pm_hbm, pm_v, sem0)
        h1 = pltpu.async_copy(a_hbm.at[pl.ds(r0, _SLAB_R)], a_v, sem1)
        h2 = pltpu.async_copy(t_hbm.at[pl.ds(r0, _SLAB_R)], t_v, sem2)
        h3 = pltpu.async_copy(val_hbm.at[pl.ds(r0, _SLAB_R)], val_v, sem3)
        h0.wait()
        h1.wait()
        h2.wait()
        h3.wait()

        acc = jnp.zeros((_L,), jnp.float32)
        for r in range(_SLAB_R):
            for cb in range(_OUT_R // _L):
                c = cb * _L
                av = a_v[r, pl.ds(c, _L)]
                tv = t_v[r, pl.ds(c, _L)]
                vv = val_v[r, pl.ds(c, _L)]
                pmv = plsc.load_gather(pm_v, [tv * _W + av])
                acc = acc + pmv * vv
        acc_v[...] = acc
        pltpu.sync_copy(acc_v, out_hbm.at[wid])

    return sc_gather


_CALLS = {}


def kernel(predict, target, penalty_matrix):
    if not _CALLS:
        _CALLS["tc"] = _make_tc_stats()
        _CALLS["sc"] = jax.jit(_make_sc_gather())
    a, t, val = _CALLS["tc"](predict, target)
    partials = _CALLS["sc"](a, t, val, penalty_matrix.reshape(-1))
    return jnp.sum(partials) / jnp.float32(predict.shape[0])
